# baseline (speedup 1.0000x reference)
import numpy as np
import jax
import jax.numpy as jnp
from jax.experimental import pallas as pl
from jax.experimental.pallas import tpu as pltpu

_DILATIONS = [1, 2, 1, 2]
_HIS_LEN = 12


def _composed_avg_matrix():
    A = np.eye(_HIS_LEN, dtype=np.float64)
    size = _HIS_LEN
    for d in _DILATIONS:
        m = size - d
        L = np.zeros((m, size))
        for i in range(m):
            L[i, i] = 0.5
            L[i, i + d] = 0.5
        A = L @ A
        size = m
    return A.astype(np.float32)


_A = _composed_avg_matrix()
_T = _HIS_LEN
_OUT_LEN = 12


def _fused_kernel(m_ref, bias_ref, x_ref, w_in_ref, w_out2_ref, b_out2_ref,
                  out_ref):
    x = x_ref[0]
    w_in = w_in_ref[...]
    w_out2 = w_out2_ref[...]
    b_out2 = b_out2_ref[...]
    for o in range(_OUT_LEN):
        z = m_ref[o, 0] * x[0]
        for t in range(1, _T):
            z = z + m_ref[o, t] * x[t]
        h = jnp.dot(z, w_in, preferred_element_type=jnp.float32)
        h = jnp.maximum(h + bias_ref[o], 0.0)
        y = jnp.dot(h, w_out2, preferred_element_type=jnp.float32)
        out_ref[0, o] = y + b_out2


def kernel(inputs, W_in, b_in, W_out1, b_out1, W_out2, b_out2):
    B, T, N, F = inputs.shape
    HID = W_in.shape[1]
    OUT_DIM = W_out2.shape[1]
    N2 = N // 2

    M = W_out1.T @ jnp.asarray(_A)
    bias = jnp.sum(M, axis=1, keepdims=True) * b_in[None, :] \
        + b_out1[:, None]
    bias2 = jnp.concatenate([bias, bias], axis=1)

    zf = jnp.zeros((F, HID), jnp.float32)
    w_in2 = jnp.block([[W_in, zf], [zf, W_in]])
    zh = jnp.zeros((HID, OUT_DIM), jnp.float32)
    w_out2b = jnp.block([[W_out2, zh], [zh, W_out2]])
    b_out2b = jnp.concatenate([b_out2, b_out2])[None, :]

    x2 = inputs.reshape(B, T, N2, 2 * F)

    out = pl.pallas_call(
        _fused_kernel,
        grid=(B,),
        in_specs=[
            pl.BlockSpec(memory_space=pltpu.SMEM),
            pl.BlockSpec((_OUT_LEN, 2 * HID), lambda b: (0, 0)),
            pl.BlockSpec((1, T, N2, 2 * F), lambda b: (b, 0, 0, 0)),
            pl.BlockSpec((2 * F, 2 * HID), lambda b: (0, 0)),
            pl.BlockSpec((2 * HID, 2 * OUT_DIM), lambda b: (0, 0)),
            pl.BlockSpec((1, 2 * OUT_DIM), lambda b: (0, 0)),
        ],
        out_specs=pl.BlockSpec((1, _OUT_LEN, N2, 2 * OUT_DIM),
                               lambda b: (b, 0, 0, 0)),
        out_shape=jax.ShapeDtypeStruct((B, _OUT_LEN, N2, 2 * OUT_DIM),
                                       jnp.float32),
        compiler_params=pltpu.CompilerParams(
            dimension_semantics=("parallel",)),
    )(M, bias2, x2, w_in2, w_out2b, b_out2b)
    return out.reshape(B, _OUT_LEN, N, OUT_DIM)


# speedup vs baseline: 1.2599x; 1.2599x over previous
import numpy as np
import jax
import jax.numpy as jnp
from jax.experimental import pallas as pl
from jax.experimental.pallas import tpu as pltpu

_DILATIONS = [1, 2, 1, 2]
_HIS_LEN = 12


def _temporal_mix_matrix():
    A = np.eye(_HIS_LEN, dtype=np.float32)
    for d in _DILATIONS:
        n = A.shape[0]
        rows = []
        left, right = 0, d
        while right <= n - 1:
            rows.append(0.5 * (A[left] + A[right]))
            left += 1
            right += 1
        A = np.stack(rows, axis=0)
    return A


def _fused_kernel(x_ref, mt_ref, w_in_ref, bias1_ref, w_out2_ref, b_out2_ref,
                  out_ref):
    T = x_ref.shape[1]
    O = mt_ref.shape[0]
    mt = mt_ref[...]
    b_out2 = b_out2_ref[...]
    for o in range(O):
        acc = x_ref[0, 0] * mt[o, 0]
        for t in range(1, T):
            acc = acc + x_ref[0, t] * mt[o, t]
        h = jax.lax.dot(acc, w_in_ref[...],
                        preferred_element_type=jnp.float32)
        h = jnp.maximum(h + bias1_ref[o], 0.0)
        y = jax.lax.dot(h, w_out2_ref[...],
                        preferred_element_type=jnp.float32)
        out_ref[0, o] = y + b_out2
    del T


def kernel(inputs, W_in, b_in, W_out1, b_out1, W_out2, b_out2):
    B, T, N, F = inputs.shape
    H = W_in.shape[1]
    O = W_out1.shape[1]
    A = jnp.asarray(_temporal_mix_matrix())
    Mt = W_out1.T @ A
    beta = jnp.sum(Mt, axis=1)
    bias1 = beta[:, None] * b_in[None, :] + b_out1[:, None]
    out = pl.pallas_call(
        _fused_kernel,
        grid=(B,),
        in_specs=[
            pl.BlockSpec((1, T, N, F), lambda b: (b, 0, 0, 0)),
            pl.BlockSpec((O, T), lambda b: (0, 0)),
            pl.BlockSpec((F, H), lambda b: (0, 0)),
            pl.BlockSpec((O, H), lambda b: (0, 0)),
            pl.BlockSpec((H, F), lambda b: (0, 0)),
            pl.BlockSpec((1, F), lambda b: (0, 0)),
        ],
        out_specs=pl.BlockSpec((1, O, N, F), lambda b: (b, 0, 0, 0)),
        out_shape=jax.ShapeDtypeStruct((B, O, N, F), jnp.float32),
        compiler_params=pltpu.CompilerParams(
            dimension_semantics=("arbitrary",)),
    )(inputs, Mt, W_in, bias1, W_out2, b_out2.reshape(1, F))
    return out


# speedup vs baseline: 1.4150x; 1.1231x over previous
import numpy as np
import jax
import jax.numpy as jnp
from jax.experimental import pallas as pl
from jax.experimental.pallas import tpu as pltpu

_DILATIONS = [1, 2, 1, 2]


def _fused_kernel(x_ref, mt_ref, w_in_ref, bias1_ref, w_out2_ref, b_out2_ref,
                  out_ref):
    T, N, F = x_ref.shape[1], x_ref.shape[2], x_ref.shape[3]
    O, S = mt_ref.shape
    H = w_in_ref.shape[1]
    s = [x_ref[0, t] for t in range(T)]
    for d in _DILATIONS:
        s = [s[i] + s[i + d] for i in range(len(s) - d)]
    mt = mt_ref[...]
    pre = []
    for o in range(O):
        acc = s[0] * mt[o, 0]
        for i in range(1, S):
            acc = acc + s[i] * mt[o, i]
        pre.append(acc)
    mixed = jnp.stack(pre, axis=0).reshape(O * N, F)
    h = jax.lax.dot(mixed, w_in_ref[...],
                    preferred_element_type=jnp.float32)
    h = h.reshape(O, N, H) + bias1_ref[...][:, None, :]
    h = jnp.maximum(h, 0.0).reshape(O * N, H)
    y = jax.lax.dot(h, w_out2_ref[...],
                    preferred_element_type=jnp.float32)
    y = y + b_out2_ref[...]
    out_ref[0] = y.reshape(O, N, F)


def kernel(inputs, W_in, b_in, W_out1, b_out1, W_out2, b_out2):
    B, T, N, F = inputs.shape
    H = W_in.shape[1]
    S, O = W_out1.shape
    Mt = W_out1.T * np.float32(0.5 ** len(_DILATIONS))
    beta = jnp.sum(W_out1, axis=0)
    bias1 = beta[:, None] * b_in[None, :] + b_out1[:, None]
    out = pl.pallas_call(
        _fused_kernel,
        grid=(B,),
        in_specs=[
            pl.BlockSpec((1, T, N, F), lambda b: (b, 0, 0, 0)),
            pl.BlockSpec((O, S), lambda b: (0, 0)),
            pl.BlockSpec((F, H), lambda b: (0, 0)),
            pl.BlockSpec((O, H), lambda b: (0, 0)),
            pl.BlockSpec((H, F), lambda b: (0, 0)),
            pl.BlockSpec((1, F), lambda b: (0, 0)),
        ],
        out_specs=pl.BlockSpec((1, O, N, F), lambda b: (b, 0, 0, 0)),
        out_shape=jax.ShapeDtypeStruct((B, O, N, F), jnp.float32),
        compiler_params=pltpu.CompilerParams(
            dimension_semantics=("arbitrary",)),
    )(inputs, Mt, W_in, bias1, W_out2, b_out2.reshape(1, F))
    return out
